# baseline (speedup 1.0000x reference)
import functools

import jax
import jax.numpy as jnp
from jax import lax
from jax.experimental import pallas as pl
from jax.experimental.pallas import tpu as pltpu
from jax.experimental.pallas import tpu_sc as plsc

N = 10000
E = 320000
F = 128
HF = 64
B = 128

NW = 32
CHUNK = 80
NCHUNK = E // (NW * CHUNK)
CH2 = 50
NCHUNK2 = E // (16 * CH2)
STAGE = (200, 200)
NBUF = 8
SL = 624
TAIL = N - 16 * SL
RB = B // 16

_mesh = plsc.VectorSubcoreMesh(core_axis_name="c", subcore_axis_name="s")


def _slice_copy(src, dst, sid):
    pltpu.sync_copy(src.at[pl.ds(sid * SL, SL)], dst.at[pl.ds(sid * SL, SL)])

    @pl.when(sid == 15)
    def _():
        pltpu.sync_copy(src.at[pl.ds(16 * SL, TAIL)],
                        dst.at[pl.ds(16 * SL, TAIL)])


@functools.partial(
    pl.kernel,
    out_type=(jax.ShapeDtypeStruct((2, N, 16), jnp.float32),
              jax.ShapeDtypeStruct((B, F), jnp.float32)),
    mesh=_mesh,
    scratch_types=[
        pltpu.VMEM((NCHUNK, CHUNK), jnp.int32),
        pltpu.VMEM((CHUNK, 16), jnp.float32),
        pltpu.VMEM_SHARED((N, 16), jnp.float32),
        pltpu.VMEM((RB,), jnp.int32),
        pltpu.VMEM((RB, F), jnp.float32),
        pltpu.SemaphoreType.DMA,
        pltpu.SemaphoreType.DMA,
    ],
)
def _sc_deg_roots(x_hbm, dst_hbm, ridx_hbm, z16_hbm, ones_hbm,
                  degp_hbm, roots_hbm,
                  dst_v, ones_v, acc_sh, ridx_v, rrow_v, rsem, sem):
    cid = lax.axis_index("c")
    sid = lax.axis_index("s")
    wid = cid * 16 + sid
    pltpu.sync_copy(dst_hbm.at[wid], dst_v)
    pltpu.sync_copy(ones_hbm, ones_v)
    _slice_copy(z16_hbm, acc_sh, sid)
    plsc.subcore_barrier()

    @pl.when(cid == 0)
    def _():
        pltpu.sync_copy(ridx_hbm.at[pl.ds(sid * RB, RB)], ridx_v)
        pltpu.async_copy(x_hbm.at[ridx_v], rrow_v, rsem).wait()
        pltpu.sync_copy(rrow_v, roots_hbm.at[pl.ds(sid * RB, RB)])

    for i in range(8):
        pltpu.async_copy(ones_v, acc_sh.at[dst_v.at[i]], sem, add=True)

    @pl.loop(8, NCHUNK)
    def _(j):
        pltpu.make_async_copy(ones_v, acc_sh.at[dst_v.at[j - 8]],
                              sem).wait()
        pltpu.async_copy(ones_v, acc_sh.at[dst_v.at[j]], sem, add=True)

    @pl.loop(0, 8)
    def _(j):
        pltpu.make_async_copy(ones_v, acc_sh.at[dst_v.at[NCHUNK - 8 + j]],
                              sem).wait()

    plsc.subcore_barrier()
    _slice_copy(acc_sh, degp_hbm.at[cid], sid)


def _make_sc_agg(with_roots):
    out_type = [jax.ShapeDtypeStruct((N, F), jnp.float32)]
    scratch = (
        [pltpu.VMEM((STAGE[0], CH2), jnp.int32),
         pltpu.VMEM((STAGE[0], CH2), jnp.int32)]
        + [pltpu.VMEM((CH2, HF), jnp.float32)] * NBUF
        + [pltpu.VMEM_SHARED((N, HF), jnp.float32)]
        + [pltpu.SemaphoreType.DMA] * (2 * NBUF)
    )
    if with_roots:
        out_type.append(jax.ShapeDtypeStruct((B, F), jnp.float32))
        scratch += [pltpu.VMEM((RB,), jnp.int32),
                    pltpu.VMEM((RB, F), jnp.float32),
                    pltpu.SemaphoreType.DMA]

    def body(refs):
        if with_roots:
            (g_hbm, src_hbm, dst_hbm, z_hbm, tab_hbm, ridx_hbm,
             agg_hbm, roots_hbm) = refs[:8]
            rest = refs[8:]
            ridx_v, rrow_v, sem2 = rest[-3:]
            rest = rest[:-3]
        else:
            (g_hbm, src_hbm, dst_hbm, z_hbm, agg_hbm) = refs[:5]
            rest = refs[5:]
        src_v, dst_v = rest[0], rest[1]
        bufs = rest[2:2 + NBUF]
        acc_sh = rest[2 + NBUF]
        gsems = rest[3 + NBUF:3 + 2 * NBUF]
        ssems = rest[3 + 2 * NBUF:3 + 3 * NBUF]
        cid = lax.axis_index("c")
        sid = lax.axis_index("s")
        gtab = g_hbm
        _slice_copy(z_hbm, acc_sh, sid)
        plsc.subcore_barrier()

        if with_roots:
            @pl.when(cid == 0)
            def _():
                pltpu.sync_copy(ridx_hbm.at[pl.ds(sid * RB, RB)], ridx_v)
                pltpu.async_copy(tab_hbm.at[ridx_v], rrow_v, sem2).wait()
                pltpu.sync_copy(rrow_v, roots_hbm.at[pl.ds(sid * RB, RB)])

        def wait_g(i, j):
            pltpu.make_async_copy(gtab.at[src_v.at[j]], bufs[i],
                                  gsems[i]).wait()

        def wait_s(i, j):
            pltpu.make_async_copy(bufs[i], acc_sh.at[dst_v.at[j]],
                                  ssems[i]).wait()

        off = 0
        for ln in STAGE:
            pltpu.sync_copy(src_hbm.at[cid, sid, pl.ds(off, ln)],
                            src_v.at[pl.ds(0, ln)])
            pltpu.sync_copy(dst_hbm.at[sid, pl.ds(off, ln)],
                            dst_v.at[pl.ds(0, ln)])

            for i in range(NBUF):
                pltpu.async_copy(gtab.at[src_v.at[i]], bufs[i], gsems[i])

            @pl.loop(0, ln - NBUF, step=NBUF)
            def _(j):
                for i in range(NBUF):
                    wait_g(i, j + i)
                    pltpu.async_copy(bufs[i], acc_sh.at[dst_v.at[j + i]],
                                     ssems[i], add=True)
                for i in range(NBUF):
                    wait_s(i, j + i)
                    pltpu.async_copy(gtab.at[src_v.at[j + NBUF + i]],
                                     bufs[i], gsems[i])

            for i in range(NBUF):
                wait_g(i, ln - NBUF + i)
                pltpu.async_copy(bufs[i], acc_sh.at[dst_v.at[ln - NBUF + i]],
                                 ssems[i], add=True)
            for i in range(NBUF):
                wait_s(i, ln - NBUF + i)

            off += ln

        plsc.subcore_barrier()
        pltpu.sync_copy(
            acc_sh.at[pl.ds(sid * SL, SL)],
            agg_hbm.at[pl.ds(sid * SL, SL), pl.ds(cid * HF, HF)])

        @pl.when(sid == 15)
        def _():
            pltpu.sync_copy(
                acc_sh.at[pl.ds(16 * SL, TAIL)],
                agg_hbm.at[pl.ds(16 * SL, TAIL), pl.ds(cid * HF, HF)])

    def entry(*refs):
        body(refs)

    return pl.kernel(entry,
                     out_type=tuple(out_type) if with_roots else out_type[0],
                     mesh=_mesh, scratch_types=scratch,
                     compiler_params=pltpu.CompilerParams(
                         use_tc_tiling_on_sc=False))


_sc_agg = _make_sc_agg(False)
_sc_agg_roots = _make_sc_agg(True)


BM = 1000


def _mm_body(x_ref, w_ref, o_ref):
    o_ref[...] = jnp.dot(x_ref[...], w_ref[...],
                         preferred_element_type=jnp.float32)


def _matmul(x, w):
    n, k = x.shape
    m = w.shape[1]
    return pl.pallas_call(
        _mm_body,
        grid=(n // BM,),
        in_specs=[pl.BlockSpec((BM, k), lambda i: (i, 0)),
                  pl.BlockSpec((k, m), lambda i: (0, 0))],
        out_specs=pl.BlockSpec((BM, m), lambda i: (i, 0)),
        out_shape=jax.ShapeDtypeStruct((n, m), jnp.float32),
    )(x, w)


def _k2_body(degp_ref, h1m_ref, dinv_ref, g1_ref):
    v = degp_ref[...]
    deg = v[0, :, 0:1] + v[1, :, 0:1] + 1.0
    dinv = lax.rsqrt(deg)
    dinv_ref[...] = dinv
    g1_ref[...] = dinv * h1m_ref[...]


def _k2(degp, h1m):
    return pl.pallas_call(
        _k2_body,
        grid=(N // BM,),
        in_specs=[pl.BlockSpec((2, BM, 16), lambda i: (0, i, 0)),
                  pl.BlockSpec((BM, F), lambda i: (i, 0))],
        out_specs=[pl.BlockSpec((BM, 1), lambda i: (i, 0)),
                   pl.BlockSpec((BM, F), lambda i: (i, 0))],
        out_shape=[jax.ShapeDtypeStruct((N, 1), jnp.float32),
                   jax.ShapeDtypeStruct((N, F), jnp.float32)],
    )(degp, h1m)


def _k3_body(aggp_ref, h1m_ref, dinv_ref, b1_ref, batch_ref, w2a_ref,
             roots_ref, w2b_ref, h1b_ref, h2m_ref, g2_ref):
    agg = aggp_ref[...]
    dinv = dinv_ref[...]
    h1m = h1m_ref[...]
    h1b = dinv * agg + (dinv * dinv) * h1m + b1_ref[...]
    h1b_ref[...] = h1b
    r1 = jax.nn.relu(h1b)
    q = jnp.dot(jax.nn.relu(roots_ref[...]), w2b_ref[...],
                preferred_element_type=jnp.float32)
    oh = (batch_ref[...] ==
          lax.broadcasted_iota(jnp.int32, (BM, B), 1)).astype(jnp.float32)
    h2m = (jnp.dot(r1, w2a_ref[...], preferred_element_type=jnp.float32)
           + jnp.dot(oh, q, preferred_element_type=jnp.float32))
    h2m_ref[...] = h2m
    g2_ref[...] = dinv * h2m


def _k3(aggp, h1m, dinv, b1r, batch2d, w2a, roots1, w2b):
    return pl.pallas_call(
        _k3_body,
        grid=(N // BM,),
        in_specs=[pl.BlockSpec((BM, F), lambda i: (i, 0)),
                  pl.BlockSpec((BM, F), lambda i: (i, 0)),
                  pl.BlockSpec((BM, 1), lambda i: (i, 0)),
                  pl.BlockSpec((1, F), lambda i: (0, 0)),
                  pl.BlockSpec((BM, 1), lambda i: (i, 0)),
                  pl.BlockSpec((F, F), lambda i: (0, 0)),
                  pl.BlockSpec((B, F), lambda i: (0, 0)),
                  pl.BlockSpec((F, F), lambda i: (0, 0))],
        out_specs=[pl.BlockSpec((BM, F), lambda i: (i, 0)),
                   pl.BlockSpec((BM, F), lambda i: (i, 0)),
                   pl.BlockSpec((BM, F), lambda i: (i, 0))],
        out_shape=[jax.ShapeDtypeStruct((N, F), jnp.float32),
                   jax.ShapeDtypeStruct((N, F), jnp.float32),
                   jax.ShapeDtypeStruct((N, F), jnp.float32)],
    )(aggp, h1m, dinv, b1r, batch2d, w2a, roots1, w2b)


def _k5_body(aggp_ref, h2m_ref, dinv_ref, b2_ref, batch_ref, x2r_ref,
             o_ref, sums_ref, cnt_ref):
    i = pl.program_id(0)
    agg = aggp_ref[...]
    dinv = dinv_ref[...]
    h2b = dinv * agg + (dinv * dinv) * h2m_ref[...] + b2_ref[...]
    r2 = jax.nn.relu(h2b)
    oh = (batch_ref[...] ==
          lax.broadcasted_iota(jnp.int32, (BM, B), 1)).astype(jnp.float32)
    part = lax.dot_general(oh, r2, (((0,), (0,)), ((), ())),
                           preferred_element_type=jnp.float32)
    ones = jnp.ones((BM, 1), jnp.float32)
    cntp = lax.dot_general(oh, ones, (((0,), (0,)), ((), ())),
                           preferred_element_type=jnp.float32)

    @pl.when(i == 0)
    def _():
        sums_ref[...] = jnp.zeros_like(sums_ref)
        cnt_ref[...] = jnp.zeros_like(cnt_ref)

    sums_ref[...] += part
    cnt_ref[...] += cntp

    @pl.when(i == N // BM - 1)
    def _():
        cnt = cnt_ref[...]
        o_ref[:, :F] = sums_ref[...] / jnp.maximum(cnt, 1.0)
        o_ref[:, F:] = jnp.where(cnt > 0.0, x2r_ref[...], 0.0)


def _k5(aggp2, h2m, dinv, b2r, batch2d, x2roots):
    return pl.pallas_call(
        _k5_body,
        grid=(N // BM,),
        in_specs=[pl.BlockSpec((BM, F), lambda i: (i, 0)),
                  pl.BlockSpec((BM, F), lambda i: (i, 0)),
                  pl.BlockSpec((BM, 1), lambda i: (i, 0)),
                  pl.BlockSpec((1, F), lambda i: (0, 0)),
                  pl.BlockSpec((BM, 1), lambda i: (i, 0)),
                  pl.BlockSpec((B, F), lambda i: (0, 0))],
        out_specs=pl.BlockSpec((B, 2 * F), lambda i: (0, 0)),
        out_shape=jax.ShapeDtypeStruct((B, 2 * F), jnp.float32),
        scratch_shapes=[pltpu.VMEM((B, F), jnp.float32),
                        pltpu.VMEM((B, 1), jnp.float32)],
    )(aggp2, h2m, dinv, b2r, batch2d, x2roots)


def kernel(x, edge_index, batch, root_index, W1, b1, W2, b2):
    src = edge_index[0].astype(jnp.int32)
    dst = edge_index[1].astype(jnp.int32)
    dst3_32 = dst.reshape(NW, NCHUNK, CHUNK)
    dst3_16 = dst.reshape(16, NCHUNK2, CH2)
    src_sc = jnp.stack([src + src, src + src + 1]).reshape(2, 16, NCHUNK2, CH2)
    batch2d = batch.astype(jnp.int32).reshape(N, 1)
    ridx = root_index.astype(jnp.int32)
    b1r = b1.reshape(1, F)
    b2r = b2.reshape(1, F)
    w2a = W2[:F, :]
    w2b = W2[F:, :]
    z64 = jnp.zeros((N, HF), jnp.float32)
    z16 = jnp.zeros((N, 16), jnp.float32)
    ones16 = jnp.ones((CHUNK, 16), jnp.float32)

    x = x.astype(jnp.float32)
    h1m = _matmul(x, W1)
    degp, roots1 = _sc_deg_roots(x, dst3_32, ridx, z16, ones16)
    src_sc, _ = lax.optimization_barrier((src_sc, h1m))
    dinv, g1 = _k2(degp, h1m)
    aggp1 = _sc_agg(g1.reshape(2 * N, HF), src_sc, dst3_16, z64)
    h1b, h2m, g2 = _k3(aggp1, h1m, dinv, b1r, batch2d, w2a,
                       roots1, w2b)
    aggp2, x2roots = _sc_agg_roots(g2.reshape(2 * N, HF), src_sc, dst3_16,
                                   z64, h1b, ridx)
    return _k5(aggp2, h2m, dinv, b2r, batch2d, x2roots)


# speedup vs baseline: 1.0745x; 1.0745x over previous
import functools

import jax
import jax.numpy as jnp
from jax import lax
from jax.experimental import pallas as pl
from jax.experimental.pallas import tpu as pltpu
from jax.experimental.pallas import tpu_sc as plsc

N = 10000
E = 320000
F = 128
HF = 64
B = 128

NW = 32
CHUNK = 80
NCHUNK = E // (NW * CHUNK)
CH2 = 125
NCHUNK2 = E // (16 * CH2)
STAGE = (80, 80)
NBUF = 4
SL = 624
TAIL = N - 16 * SL
RB = B // 16

_mesh = plsc.VectorSubcoreMesh(core_axis_name="c", subcore_axis_name="s")


def _slice_copy(src, dst, sid):
    pltpu.sync_copy(src.at[pl.ds(sid * SL, SL)], dst.at[pl.ds(sid * SL, SL)])

    @pl.when(sid == 15)
    def _():
        pltpu.sync_copy(src.at[pl.ds(16 * SL, TAIL)],
                        dst.at[pl.ds(16 * SL, TAIL)])


@functools.partial(
    pl.kernel,
    out_type=(jax.ShapeDtypeStruct((2, N, 16), jnp.float32),
              jax.ShapeDtypeStruct((B, F), jnp.float32)),
    mesh=_mesh,
    scratch_types=[
        pltpu.VMEM((NCHUNK, CHUNK), jnp.int32),
        pltpu.VMEM((CHUNK, 16), jnp.float32),
        pltpu.VMEM_SHARED((N, 16), jnp.float32),
        pltpu.VMEM((RB,), jnp.int32),
        pltpu.VMEM((RB, F), jnp.float32),
        pltpu.SemaphoreType.DMA,
        pltpu.SemaphoreType.DMA,
    ],
)
def _sc_deg_roots(x_hbm, dst_hbm, ridx_hbm, z16_hbm, ones_hbm,
                  degp_hbm, roots_hbm,
                  dst_v, ones_v, acc_sh, ridx_v, rrow_v, rsem, sem):
    cid = lax.axis_index("c")
    sid = lax.axis_index("s")
    wid = cid * 16 + sid
    pltpu.sync_copy(dst_hbm.at[wid], dst_v)
    pltpu.sync_copy(ones_hbm, ones_v)
    _slice_copy(z16_hbm, acc_sh, sid)
    plsc.subcore_barrier()

    @pl.when(cid == 0)
    def _():
        pltpu.sync_copy(ridx_hbm.at[pl.ds(sid * RB, RB)], ridx_v)
        pltpu.async_copy(x_hbm.at[ridx_v], rrow_v, rsem).wait()
        pltpu.sync_copy(rrow_v, roots_hbm.at[pl.ds(sid * RB, RB)])

    for i in range(8):
        pltpu.async_copy(ones_v, acc_sh.at[dst_v.at[i]], sem, add=True)

    @pl.loop(8, NCHUNK)
    def _(j):
        pltpu.make_async_copy(ones_v, acc_sh.at[dst_v.at[j - 8]],
                              sem).wait()
        pltpu.async_copy(ones_v, acc_sh.at[dst_v.at[j]], sem, add=True)

    @pl.loop(0, 8)
    def _(j):
        pltpu.make_async_copy(ones_v, acc_sh.at[dst_v.at[NCHUNK - 8 + j]],
                              sem).wait()

    plsc.subcore_barrier()
    _slice_copy(acc_sh, degp_hbm.at[cid], sid)


def _make_sc_agg(with_roots):
    out_type = [jax.ShapeDtypeStruct((N, F), jnp.float32)]
    scratch = (
        [pltpu.VMEM((STAGE[0], CH2), jnp.int32),
         pltpu.VMEM((STAGE[0], CH2), jnp.int32)]
        + [pltpu.VMEM((CH2, HF), jnp.float32)] * NBUF
        + [pltpu.VMEM_SHARED((N, HF), jnp.float32)]
        + [pltpu.SemaphoreType.DMA] * (2 * NBUF)
    )
    if with_roots:
        out_type.append(jax.ShapeDtypeStruct((B, F), jnp.float32))
        scratch += [pltpu.VMEM((RB,), jnp.int32),
                    pltpu.VMEM((RB, F), jnp.float32),
                    pltpu.SemaphoreType.DMA]

    def body(refs):
        if with_roots:
            (g_hbm, src_hbm, dst_hbm, z_hbm, tab_hbm, ridx_hbm,
             agg_hbm, roots_hbm) = refs[:8]
            rest = refs[8:]
            ridx_v, rrow_v, sem2 = rest[-3:]
            rest = rest[:-3]
        else:
            (g_hbm, src_hbm, dst_hbm, z_hbm, agg_hbm) = refs[:5]
            rest = refs[5:]
        src_v, dst_v = rest[0], rest[1]
        bufs = rest[2:2 + NBUF]
        acc_sh = rest[2 + NBUF]
        gsems = rest[3 + NBUF:3 + 2 * NBUF]
        ssems = rest[3 + 2 * NBUF:3 + 3 * NBUF]
        cid = lax.axis_index("c")
        sid = lax.axis_index("s")
        gtab = g_hbm
        _slice_copy(z_hbm, acc_sh, sid)
        plsc.subcore_barrier()

        if with_roots:
            @pl.when(cid == 0)
            def _():
                pltpu.sync_copy(ridx_hbm.at[pl.ds(sid * RB, RB)], ridx_v)
                pltpu.async_copy(tab_hbm.at[ridx_v], rrow_v, sem2).wait()
                pltpu.sync_copy(rrow_v, roots_hbm.at[pl.ds(sid * RB, RB)])

        def wait_g(i, j):
            pltpu.make_async_copy(gtab.at[src_v.at[j]], bufs[i],
                                  gsems[i]).wait()

        def wait_s(i, j):
            pltpu.make_async_copy(bufs[i], acc_sh.at[dst_v.at[j]],
                                  ssems[i]).wait()

        off = 0
        for ln in STAGE:
            pltpu.sync_copy(src_hbm.at[cid, sid, pl.ds(off, ln)],
                            src_v.at[pl.ds(0, ln)])
            pltpu.sync_copy(dst_hbm.at[sid, pl.ds(off, ln)],
                            dst_v.at[pl.ds(0, ln)])

            for i in range(NBUF):
                pltpu.async_copy(gtab.at[src_v.at[i]], bufs[i], gsems[i])

            @pl.loop(0, ln - NBUF, step=NBUF)
            def _(j):
                for i in range(NBUF):
                    wait_g(i, j + i)
                    pltpu.async_copy(bufs[i], acc_sh.at[dst_v.at[j + i]],
                                     ssems[i], add=True)
                for i in range(NBUF):
                    wait_s(i, j + i)
                    pltpu.async_copy(gtab.at[src_v.at[j + NBUF + i]],
                                     bufs[i], gsems[i])

            for i in range(NBUF):
                wait_g(i, ln - NBUF + i)
                pltpu.async_copy(bufs[i], acc_sh.at[dst_v.at[ln - NBUF + i]],
                                 ssems[i], add=True)
            for i in range(NBUF):
                wait_s(i, ln - NBUF + i)

            off += ln

        plsc.subcore_barrier()
        pltpu.sync_copy(
            acc_sh.at[pl.ds(sid * SL, SL)],
            agg_hbm.at[pl.ds(sid * SL, SL), pl.ds(cid * HF, HF)])

        @pl.when(sid == 15)
        def _():
            pltpu.sync_copy(
                acc_sh.at[pl.ds(16 * SL, TAIL)],
                agg_hbm.at[pl.ds(16 * SL, TAIL), pl.ds(cid * HF, HF)])

    def entry(*refs):
        body(refs)

    return pl.kernel(entry,
                     out_type=tuple(out_type) if with_roots else out_type[0],
                     mesh=_mesh, scratch_types=scratch,
                     compiler_params=pltpu.CompilerParams(
                         use_tc_tiling_on_sc=False))


_sc_agg = _make_sc_agg(False)
_sc_agg_roots = _make_sc_agg(True)


BM = 1000


def _mm_body(x_ref, w_ref, o_ref):
    o_ref[...] = jnp.dot(x_ref[...], w_ref[...],
                         preferred_element_type=jnp.float32)


def _matmul(x, w):
    n, k = x.shape
    m = w.shape[1]
    return pl.pallas_call(
        _mm_body,
        grid=(n // BM,),
        in_specs=[pl.BlockSpec((BM, k), lambda i: (i, 0)),
                  pl.BlockSpec((k, m), lambda i: (0, 0))],
        out_specs=pl.BlockSpec((BM, m), lambda i: (i, 0)),
        out_shape=jax.ShapeDtypeStruct((n, m), jnp.float32),
    )(x, w)


def _k2_body(degp_ref, h1m_ref, dinv_ref, g1_ref):
    v = degp_ref[...]
    deg = v[0, :, 0:1] + v[1, :, 0:1] + 1.0
    dinv = lax.rsqrt(deg)
    dinv_ref[...] = dinv
    g1_ref[...] = dinv * h1m_ref[...]


def _k2(degp, h1m):
    return pl.pallas_call(
        _k2_body,
        grid=(N // BM,),
        in_specs=[pl.BlockSpec((2, BM, 16), lambda i: (0, i, 0)),
                  pl.BlockSpec((BM, F), lambda i: (i, 0))],
        out_specs=[pl.BlockSpec((BM, 1), lambda i: (i, 0)),
                   pl.BlockSpec((BM, F), lambda i: (i, 0))],
        out_shape=[jax.ShapeDtypeStruct((N, 1), jnp.float32),
                   jax.ShapeDtypeStruct((N, F), jnp.float32)],
    )(degp, h1m)


def _k3_body(aggp_ref, h1m_ref, dinv_ref, b1_ref, batch_ref, w2a_ref,
             roots_ref, w2b_ref, h1b_ref, h2m_ref, g2_ref):
    agg = aggp_ref[...]
    dinv = dinv_ref[...]
    h1m = h1m_ref[...]
    h1b = dinv * agg + (dinv * dinv) * h1m + b1_ref[...]
    h1b_ref[...] = h1b
    r1 = jax.nn.relu(h1b)
    q = jnp.dot(jax.nn.relu(roots_ref[...]), w2b_ref[...],
                preferred_element_type=jnp.float32)
    oh = (batch_ref[...] ==
          lax.broadcasted_iota(jnp.int32, (BM, B), 1)).astype(jnp.float32)
    h2m = (jnp.dot(r1, w2a_ref[...], preferred_element_type=jnp.float32)
           + jnp.dot(oh, q, preferred_element_type=jnp.float32))
    h2m_ref[...] = h2m
    g2_ref[...] = dinv * h2m


def _k3(aggp, h1m, dinv, b1r, batch2d, w2a, roots1, w2b):
    return pl.pallas_call(
        _k3_body,
        grid=(N // BM,),
        in_specs=[pl.BlockSpec((BM, F), lambda i: (i, 0)),
                  pl.BlockSpec((BM, F), lambda i: (i, 0)),
                  pl.BlockSpec((BM, 1), lambda i: (i, 0)),
                  pl.BlockSpec((1, F), lambda i: (0, 0)),
                  pl.BlockSpec((BM, 1), lambda i: (i, 0)),
                  pl.BlockSpec((F, F), lambda i: (0, 0)),
                  pl.BlockSpec((B, F), lambda i: (0, 0)),
                  pl.BlockSpec((F, F), lambda i: (0, 0))],
        out_specs=[pl.BlockSpec((BM, F), lambda i: (i, 0)),
                   pl.BlockSpec((BM, F), lambda i: (i, 0)),
                   pl.BlockSpec((BM, F), lambda i: (i, 0))],
        out_shape=[jax.ShapeDtypeStruct((N, F), jnp.float32),
                   jax.ShapeDtypeStruct((N, F), jnp.float32),
                   jax.ShapeDtypeStruct((N, F), jnp.float32)],
    )(aggp, h1m, dinv, b1r, batch2d, w2a, roots1, w2b)


def _k5_body(aggp_ref, h2m_ref, dinv_ref, b2_ref, batch_ref, x2r_ref,
             o_ref, sums_ref, cnt_ref):
    i = pl.program_id(0)
    agg = aggp_ref[...]
    dinv = dinv_ref[...]
    h2b = dinv * agg + (dinv * dinv) * h2m_ref[...] + b2_ref[...]
    r2 = jax.nn.relu(h2b)
    oh = (batch_ref[...] ==
          lax.broadcasted_iota(jnp.int32, (BM, B), 1)).astype(jnp.float32)
    part = lax.dot_general(oh, r2, (((0,), (0,)), ((), ())),
                           preferred_element_type=jnp.float32)
    ones = jnp.ones((BM, 1), jnp.float32)
    cntp = lax.dot_general(oh, ones, (((0,), (0,)), ((), ())),
                           preferred_element_type=jnp.float32)

    @pl.when(i == 0)
    def _():
        sums_ref[...] = jnp.zeros_like(sums_ref)
        cnt_ref[...] = jnp.zeros_like(cnt_ref)

    sums_ref[...] += part
    cnt_ref[...] += cntp

    @pl.when(i == N // BM - 1)
    def _():
        cnt = cnt_ref[...]
        o_ref[:, :F] = sums_ref[...] / jnp.maximum(cnt, 1.0)
        o_ref[:, F:] = jnp.where(cnt > 0.0, x2r_ref[...], 0.0)


def _k5(aggp2, h2m, dinv, b2r, batch2d, x2roots):
    return pl.pallas_call(
        _k5_body,
        grid=(N // BM,),
        in_specs=[pl.BlockSpec((BM, F), lambda i: (i, 0)),
                  pl.BlockSpec((BM, F), lambda i: (i, 0)),
                  pl.BlockSpec((BM, 1), lambda i: (i, 0)),
                  pl.BlockSpec((1, F), lambda i: (0, 0)),
                  pl.BlockSpec((BM, 1), lambda i: (i, 0)),
                  pl.BlockSpec((B, F), lambda i: (0, 0))],
        out_specs=pl.BlockSpec((B, 2 * F), lambda i: (0, 0)),
        out_shape=jax.ShapeDtypeStruct((B, 2 * F), jnp.float32),
        scratch_shapes=[pltpu.VMEM((B, F), jnp.float32),
                        pltpu.VMEM((B, 1), jnp.float32)],
    )(aggp2, h2m, dinv, b2r, batch2d, x2roots)


def kernel(x, edge_index, batch, root_index, W1, b1, W2, b2):
    src = edge_index[0].astype(jnp.int32)
    dst = edge_index[1].astype(jnp.int32)
    dst3_32 = dst.reshape(NW, NCHUNK, CHUNK)
    dst3_16 = dst.reshape(16, NCHUNK2, CH2)
    src_sc = jnp.stack([src + src, src + src + 1]).reshape(2, 16, NCHUNK2, CH2)
    batch2d = batch.astype(jnp.int32).reshape(N, 1)
    ridx = root_index.astype(jnp.int32)
    b1r = b1.reshape(1, F)
    b2r = b2.reshape(1, F)
    w2a = W2[:F, :]
    w2b = W2[F:, :]
    z64 = jnp.zeros((N, HF), jnp.float32)
    z16 = jnp.zeros((N, 16), jnp.float32)
    ones16 = jnp.ones((CHUNK, 16), jnp.float32)

    x = x.astype(jnp.float32)
    h1m = _matmul(x, W1)
    degp, roots1 = _sc_deg_roots(x, dst3_32, ridx, z16, ones16)
    src_sc, _ = lax.optimization_barrier((src_sc, h1m))
    dinv, g1 = _k2(degp, h1m)
    aggp1 = _sc_agg(g1.reshape(2 * N, HF), src_sc, dst3_16, z64)
    h1b, h2m, g2 = _k3(aggp1, h1m, dinv, b1r, batch2d, w2a,
                       roots1, w2b)
    aggp2, x2roots = _sc_agg_roots(g2.reshape(2 * N, HF), src_sc, dst3_16,
                                   z64, h1b, ridx)
    return _k5(aggp2, h2m, dinv, b2r, batch2d, x2roots)
